# Initial kernel scaffold; baseline (speedup 1.0000x reference)
import jax
import jax.numpy as jnp
from jax.experimental import pallas as pl


def kernel(word_table, ent_table, W, b, context_ids, cand_ids):
    raise NotImplementedError("write your pallas kernel here")


# speedup vs baseline: 1.6703x; 1.6703x over previous
import functools

import jax
import jax.numpy as jnp
from jax import lax
from jax.experimental import pallas as pl
from jax.experimental.pallas import tpu as pltpu
from jax.experimental.pallas import tpu_sc as plsc

NC = 2
NS = 16
NW = NC * NS
LANES = 16
SLOTS = 4
PRE = 2


def _emit_ring(n_b, start_gather, wait_gather, start_wb, wait_wb):
    assert n_b % 2 == 0 and n_b >= 8
    u0 = 2 if n_b % 4 == 0 else 4
    n_uniform = (n_b - 2) - u0
    assert n_uniform % 4 == 0

    start_gather(0, 0)
    start_gather(1, 1)
    for t in range(u0):
        if t >= PRE:
            wait_wb((t + PRE) % SLOTS)
        start_gather(t + PRE, (t + PRE) % SLOTS)
        wait_gather(t, t % SLOTS)
        start_wb(t, t % SLOTS)

    @pl.loop(0, n_uniform // 4)
    def _(j):
        t0 = u0 + j * 4
        for b in range(4):
            s = (u0 + b) % SLOTS
            s2 = (u0 + b + PRE) % SLOTS
            wait_wb(s2)
            start_gather(t0 + b + PRE, s2)
            wait_gather(t0 + b, s)
            start_wb(t0 + b, s)

    for t in range(n_b - 2, n_b):
        wait_gather(t, t % SLOTS)
        start_wb(t, t % SLOTS)
    for t in range(n_b - 4, n_b):
        wait_wb(t % SLOTS)


def _word_context_sum(word_table, flat_ids, seg_ids, B, L, D, GB=128):
    per_w = B // NW
    ids_per_w = per_w * L
    n_b = ids_per_w // GB
    mesh = plsc.VectorSubcoreMesh(core_axis_name="c", subcore_axis_name="s")

    @functools.partial(
        pl.kernel,
        out_type=jax.ShapeDtypeStruct((B, D), jnp.float32),
        mesh=mesh,
        compiler_params=pltpu.CompilerParams(use_tc_tiling_on_sc=False),
        scratch_types=[
            pltpu.VMEM((ids_per_w,), jnp.int32),
            pltpu.VMEM((n_b, GB), jnp.int32),
            pltpu.VMEM((SLOTS, GB, D), jnp.float32),
            pltpu.VMEM((per_w, D), jnp.float32),
            pltpu.VMEM_SHARED((NS * per_w, D), jnp.float32),
        ] + [pltpu.SemaphoreType.DMA] * (2 * SLOTS),
    )
    def k(tbl_hbm, ids_hbm, seg_hbm, out_hbm, ids_v, seg_v, rows_v, zero_v,
          acc_sh, *sems):
        sem_g, sem_w = sems[:SLOTS], sems[SLOTS:]
        c = lax.axis_index("c")
        s_idx = lax.axis_index("s")
        wid = s_idx * NC + c
        base_b = wid * per_w
        local_base = s_idx * per_w
        pltpu.sync_copy(ids_hbm.at[pl.ds(base_b * L, ids_per_w)], ids_v)
        pltpu.sync_copy(seg_hbm, seg_v)

        off = jnp.full((LANES,), local_base, jnp.int32)

        @pl.loop(0, n_b)
        def _(t):
            for j in range(GB // LANES):
                sl = pl.ds(j * LANES, LANES)
                seg_v.at[t, sl][...] = seg_v.at[t, sl][...] + off

        @pl.loop(0, per_w)
        def _(i):
            for j in range(D // LANES):
                zero_v.at[i, pl.ds(j * LANES, LANES)][...] = jnp.zeros(
                    (LANES,), jnp.float32)

        pltpu.sync_copy(zero_v, acc_sh.at[pl.ds(local_base, per_w)])

        def start_gather(t, s):
            pltpu.async_copy(
                tbl_hbm.at[ids_v.at[pl.ds(t * GB, GB)]], rows_v.at[s],
                sem_g[s])

        def wait_gather(t, s):
            pltpu.make_async_copy(
                tbl_hbm.at[ids_v.at[pl.ds(0, GB)]], rows_v.at[s],
                sem_g[s]).wait()

        def start_wb(t, s):
            pltpu.async_copy(rows_v.at[s], acc_sh.at[seg_v.at[t]], sem_w[s],
                             add=True)

        def wait_wb(s):
            pltpu.make_async_copy(rows_v.at[s], acc_sh.at[seg_v.at[0]],
                                  sem_w[s]).wait()

        _emit_ring(n_b, start_gather, wait_gather, start_wb, wait_wb)

        pltpu.sync_copy(acc_sh.at[pl.ds(local_base, per_w)],
                        out_hbm.at[pl.ds(base_b, per_w)])

    return k(word_table, flat_ids, seg_ids)


def _ent_gather(ent_table, flat_ids, N, D, GB=64):
    per_w = N // NW
    n_b = per_w // GB
    mesh = plsc.VectorSubcoreMesh(core_axis_name="c", subcore_axis_name="s")

    @functools.partial(
        pl.kernel,
        out_type=jax.ShapeDtypeStruct((N, D), jnp.float32),
        mesh=mesh,
        compiler_params=pltpu.CompilerParams(use_tc_tiling_on_sc=False),
        scratch_types=[
            pltpu.VMEM((per_w,), jnp.int32),
            pltpu.VMEM((SLOTS, GB, D), jnp.float32),
        ] + [pltpu.SemaphoreType.DMA] * (2 * SLOTS),
    )
    def k(tbl_hbm, ids_hbm, out_hbm, ids_v, rows_v, *sems):
        sem_g, sem_w = sems[:SLOTS], sems[SLOTS:]
        wid = lax.axis_index("s") * NC + lax.axis_index("c")
        base = wid * per_w
        pltpu.sync_copy(ids_hbm.at[pl.ds(base, per_w)], ids_v)

        def start_gather(t, s):
            pltpu.async_copy(
                tbl_hbm.at[ids_v.at[pl.ds(t * GB, GB)]], rows_v.at[s],
                sem_g[s])

        def wait_gather(t, s):
            pltpu.make_async_copy(
                tbl_hbm.at[ids_v.at[pl.ds(0, GB)]], rows_v.at[s],
                sem_g[s]).wait()

        def start_wb(t, s):
            pltpu.async_copy(rows_v.at[s], out_hbm.at[pl.ds(base + t * GB,
                                                            GB)], sem_w[s])

        def wait_wb(s):
            pltpu.make_async_copy(rows_v.at[s], out_hbm.at[pl.ds(base, GB)],
                                  sem_w[s]).wait()

        _emit_ring(n_b, start_gather, wait_gather, start_wb, wait_wb)

    return k(ent_table, flat_ids)


def _tc_scores(ctx_sum, Wt_scaled, bias, ent_rows, B, C, D, blk=256):
    grid = (B // blk,)

    def body(m_ref, wt_ref, b_ref, ent_ref, out_ref):
        ctx = jnp.dot(m_ref[...], wt_ref[...],
                      preferred_element_type=jnp.float32) + b_ref[...]
        cand = ent_ref[...].reshape(blk, C, D)
        out_ref[...] = jnp.sum(cand * ctx[:, None, :], axis=-1)

    return pl.pallas_call(
        body,
        grid=grid,
        in_specs=[
            pl.BlockSpec((blk, D), lambda i: (i, 0)),
            pl.BlockSpec((D, D), lambda i: (0, 0)),
            pl.BlockSpec((1, D), lambda i: (0, 0)),
            pl.BlockSpec((blk * C, D), lambda i: (i, 0)),
        ],
        out_specs=pl.BlockSpec((blk, C), lambda i: (i, 0)),
        out_shape=jax.ShapeDtypeStruct((B, C), jnp.float32),
    )(ctx_sum, Wt_scaled, bias, ent_rows)


def kernel(word_table, ent_table, W, b, context_ids, cand_ids):
    B, L = context_ids.shape
    _, C = cand_ids.shape
    D = word_table.shape[1]

    flat_ctx = context_ids.reshape(-1).astype(jnp.int32)
    flat_cand = cand_ids.reshape(-1).astype(jnp.int32)
    ids_per_w = (B // NW) * L
    seg_ids = (jnp.arange(ids_per_w, dtype=jnp.int32) // L).reshape(
        ids_per_w // 128, 128)

    ctx_sum = _word_context_sum(word_table, flat_ctx, seg_ids, B, L, D)
    ent_rows = _ent_gather(ent_table, flat_cand, B * C, D)

    Wt_scaled = (W.T / L).astype(jnp.float32)
    bias = b.reshape(1, D).astype(jnp.float32)
    return _tc_scores(ctx_sum, Wt_scaled, bias, ent_rows, B, C, D)


# speedup vs baseline: 1.6758x; 1.0033x over previous
import functools

import jax
import jax.numpy as jnp
from jax import lax
from jax.experimental import pallas as pl
from jax.experimental.pallas import tpu as pltpu
from jax.experimental.pallas import tpu_sc as plsc

NC = 2
NS = 16
NW = NC * NS
LANES = 16
SLOTS = 4
PRE = 2


def _emit_ring(n_b, start_gather, wait_gather, start_wb, wait_wb):
    assert n_b % 2 == 0 and n_b >= 8
    u0 = 2 if n_b % 4 == 0 else 4
    n_uniform = (n_b - 2) - u0
    assert n_uniform % 4 == 0

    start_gather(0, 0)
    start_gather(1, 1)
    for t in range(u0):
        if t >= PRE:
            wait_wb((t + PRE) % SLOTS)
        start_gather(t + PRE, (t + PRE) % SLOTS)
        wait_gather(t, t % SLOTS)
        start_wb(t, t % SLOTS)

    @pl.loop(0, n_uniform // 4)
    def _(j):
        t0 = u0 + j * 4
        for b in range(4):
            s = (u0 + b) % SLOTS
            s2 = (u0 + b + PRE) % SLOTS
            wait_wb(s2)
            start_gather(t0 + b + PRE, s2)
            wait_gather(t0 + b, s)
            start_wb(t0 + b, s)

    for t in range(n_b - 2, n_b):
        wait_gather(t, t % SLOTS)
        start_wb(t, t % SLOTS)
    for t in range(n_b - 4, n_b):
        wait_wb(t % SLOTS)


def _word_context_sum(word_table, flat_ids, seg_ids, B, L, D, GB=128):
    per_w = B // NW
    ids_per_w = per_w * L
    n_b = ids_per_w // GB
    mesh = plsc.VectorSubcoreMesh(core_axis_name="c", subcore_axis_name="s")

    @functools.partial(
        pl.kernel,
        out_type=jax.ShapeDtypeStruct((B, D), jnp.float32),
        mesh=mesh,
        compiler_params=pltpu.CompilerParams(use_tc_tiling_on_sc=False),
        scratch_types=[
            pltpu.VMEM((ids_per_w,), jnp.int32),
            pltpu.VMEM((n_b, GB), jnp.int32),
            pltpu.VMEM((SLOTS, GB, D), jnp.float32),
            pltpu.VMEM((per_w, D), jnp.float32),
            pltpu.VMEM_SHARED((NS * per_w, D), jnp.float32),
        ] + [pltpu.SemaphoreType.DMA] * (2 * SLOTS),
    )
    def k(tbl_hbm, ids_hbm, seg_hbm, out_hbm, ids_v, seg_v, rows_v, zero_v,
          acc_sh, *sems):
        sem_g, sem_w = sems[:SLOTS], sems[SLOTS:]
        c = lax.axis_index("c")
        s_idx = lax.axis_index("s")
        wid = s_idx * NC + c
        base_b = wid * per_w
        local_base = s_idx * per_w
        pltpu.sync_copy(ids_hbm.at[pl.ds(base_b * L, ids_per_w)], ids_v)
        pltpu.sync_copy(seg_hbm, seg_v)

        off = jnp.full((LANES,), local_base, jnp.int32)

        @pl.loop(0, n_b)
        def _(t):
            for j in range(GB // LANES):
                sl = pl.ds(j * LANES, LANES)
                seg_v.at[t, sl][...] = seg_v.at[t, sl][...] + off

        @pl.loop(0, per_w)
        def _(i):
            for j in range(D // LANES):
                zero_v.at[i, pl.ds(j * LANES, LANES)][...] = jnp.zeros(
                    (LANES,), jnp.float32)

        pltpu.sync_copy(zero_v, acc_sh.at[pl.ds(local_base, per_w)])

        def start_gather(t, s):
            pltpu.async_copy(
                tbl_hbm.at[ids_v.at[pl.ds(t * GB, GB)]], rows_v.at[s],
                sem_g[s])

        def wait_gather(t, s):
            pltpu.make_async_copy(
                tbl_hbm.at[ids_v.at[pl.ds(0, GB)]], rows_v.at[s],
                sem_g[s]).wait()

        def start_wb(t, s):
            pltpu.async_copy(rows_v.at[s], acc_sh.at[seg_v.at[t]], sem_w[s],
                             add=True)

        def wait_wb(s):
            pltpu.make_async_copy(rows_v.at[s], acc_sh.at[seg_v.at[0]],
                                  sem_w[s]).wait()

        _emit_ring(n_b, start_gather, wait_gather, start_wb, wait_wb)

        pltpu.sync_copy(acc_sh.at[pl.ds(local_base, per_w)],
                        out_hbm.at[pl.ds(base_b, per_w)])

    return k(word_table, flat_ids, seg_ids)


def _ent_gather(ent_table, flat_ids, N, D, GB=416):
    per_w = N // NW
    n_b = per_w // GB
    mesh = plsc.VectorSubcoreMesh(core_axis_name="c", subcore_axis_name="s")

    @functools.partial(
        pl.kernel,
        out_type=jax.ShapeDtypeStruct((N, D), jnp.float32),
        mesh=mesh,
        compiler_params=pltpu.CompilerParams(use_tc_tiling_on_sc=False),
        scratch_types=[
            pltpu.VMEM((per_w,), jnp.int32),
            pltpu.VMEM((SLOTS, GB, D), jnp.float32),
        ] + [pltpu.SemaphoreType.DMA] * (2 * SLOTS),
    )
    def k(tbl_hbm, ids_hbm, out_hbm, ids_v, rows_v, *sems):
        sem_g, sem_w = sems[:SLOTS], sems[SLOTS:]
        wid = lax.axis_index("s") * NC + lax.axis_index("c")
        base = wid * per_w
        pltpu.sync_copy(ids_hbm.at[pl.ds(base, per_w)], ids_v)

        def start_gather(t, s):
            pltpu.async_copy(
                tbl_hbm.at[ids_v.at[pl.ds(t * GB, GB)]], rows_v.at[s],
                sem_g[s])

        def wait_gather(t, s):
            pltpu.make_async_copy(
                tbl_hbm.at[ids_v.at[pl.ds(0, GB)]], rows_v.at[s],
                sem_g[s]).wait()

        def start_wb(t, s):
            pltpu.async_copy(rows_v.at[s], out_hbm.at[pl.ds(base + t * GB,
                                                            GB)], sem_w[s])

        def wait_wb(s):
            pltpu.make_async_copy(rows_v.at[s], out_hbm.at[pl.ds(base, GB)],
                                  sem_w[s]).wait()

        _emit_ring(n_b, start_gather, wait_gather, start_wb, wait_wb)

    return k(ent_table, flat_ids)


def _tc_scores(ctx_sum, Wt_scaled, bias, ent_rows, B, C, D, blk=256):
    grid = (B // blk,)

    def body(m_ref, wt_ref, b_ref, ent_ref, out_ref):
        ctx = jnp.dot(m_ref[...], wt_ref[...],
                      preferred_element_type=jnp.float32) + b_ref[...]
        cand = ent_ref[...].reshape(blk, C, D)
        out_ref[...] = jnp.sum(cand * ctx[:, None, :], axis=-1)

    return pl.pallas_call(
        body,
        grid=grid,
        in_specs=[
            pl.BlockSpec((blk, D), lambda i: (i, 0)),
            pl.BlockSpec((D, D), lambda i: (0, 0)),
            pl.BlockSpec((1, D), lambda i: (0, 0)),
            pl.BlockSpec((blk * C, D), lambda i: (i, 0)),
        ],
        out_specs=pl.BlockSpec((blk, C), lambda i: (i, 0)),
        out_shape=jax.ShapeDtypeStruct((B, C), jnp.float32),
    )(ctx_sum, Wt_scaled, bias, ent_rows)


def kernel(word_table, ent_table, W, b, context_ids, cand_ids):
    B, L = context_ids.shape
    _, C = cand_ids.shape
    D = word_table.shape[1]

    flat_ctx = context_ids.reshape(-1).astype(jnp.int32)
    flat_cand = cand_ids.reshape(-1).astype(jnp.int32)
    ids_per_w = (B // NW) * L
    seg_ids = (jnp.arange(ids_per_w, dtype=jnp.int32) // L).reshape(
        ids_per_w // 128, 128)

    ctx_sum = _word_context_sum(word_table, flat_ctx, seg_ids, B, L, D)
    ent_rows = _ent_gather(ent_table, flat_cand, B * C, D)

    Wt_scaled = (W.T / L).astype(jnp.float32)
    bias = b.reshape(1, D).astype(jnp.float32)
    return _tc_scores(ctx_sum, Wt_scaled, bias, ent_rows, B, C, D)
